# Initial kernel scaffold; baseline (speedup 1.0000x reference)
import jax
import jax.numpy as jnp
from jax.experimental import pallas as pl


def kernel(x, edge_index, shuffled_index, W1, b1, W2, b2, Wd, bd):
    raise NotImplementedError("write your pallas kernel here")


# speedup vs baseline: 4.9902x; 4.9902x over previous
import dataclasses
import functools

import jax
import jax.numpy as jnp
from jax.experimental import pallas as pl
from jax.experimental.pallas import tpu as pltpu
from jax.experimental.pallas import tpu_sc as plsc

N = 10000
D = 128
E = 320000
NC, NS = 2, 16
NW = NC * NS
N_PAD = 10112
RPS = N_PAD // NS
EPT = 10112
CHUNK = 128
NCHUNK = EPT // CHUNK

SHUF_PAD = 10240
S_CHUNK = 64
S_NCHUNK = 5
SPT = S_CHUNK * S_NCHUNK

_mesh = lambda: plsc.VectorSubcoreMesh(core_axis_name="c", subcore_axis_name="s")


def _sc_agg():
    def body(h_hbm, src_hbm, dst_hbm, agg_out,
             src_v, dst_v, rows_v, agg_sp):
        cid = jax.lax.axis_index("c")
        sid = jax.lax.axis_index("s")
        wid = cid * NS + sid
        rsl = pl.ds(sid * RPS, RPS)
        pltpu.sync_copy(h_hbm.at[rsl], agg_sp.at[rsl])
        pltpu.sync_copy(src_hbm.at[wid], src_v)
        pltpu.sync_copy(dst_hbm.at[wid], dst_v)
        plsc.subcore_barrier()

        @pl.loop(0, NCHUNK)
        def _(j):
            pltpu.sync_copy(h_hbm.at[src_v.at[j]], rows_v)
            pltpu.sync_copy(rows_v, agg_sp.at[dst_v.at[j]], add=True)

        plsc.subcore_barrier()
        pltpu.sync_copy(agg_sp.at[rsl], agg_out.at[cid, rsl])

    return pl.kernel(
        body,
        out_type=jax.ShapeDtypeStruct((NC, N_PAD, D), jnp.float32),
        mesh=_mesh(),
        scratch_types=[
            pltpu.VMEM((NCHUNK, CHUNK), jnp.int32),
            pltpu.VMEM((NCHUNK, CHUNK), jnp.int32),
            pltpu.VMEM((CHUNK, D), jnp.float32),
            pltpu.VMEM_SHARED((N_PAD, D), jnp.float32),
        ],
    )


def _sc_deg():
    def body(dst_hbm, deg_out, dst_v, hist_v):
        cid = jax.lax.axis_index("c")
        sid = jax.lax.axis_index("s")
        wid = cid * NS + sid
        pltpu.sync_copy(dst_hbm.at[wid], dst_v)

        @pl.loop(0, N_PAD // 16)
        def _(i):
            hist_v[pl.ds(i * 16, 16)] = jnp.zeros((16,), jnp.float32)

        ones16 = jnp.ones((16,), jnp.float32)

        @pl.loop(0, NCHUNK)
        def _(j):
            @pl.loop(0, CHUNK // 16)
            def _(k):
                idx = dst_v[j, pl.ds(k * 16, 16)]
                plsc.addupdate_scatter(hist_v, [idx], ones16)

        pltpu.sync_copy(hist_v, deg_out.at[wid])

    cp = pltpu.CompilerParams()
    if "needs_layout_passes" in pltpu.CompilerParams.__dataclass_fields__:
        cp = dataclasses.replace(cp, needs_layout_passes=False)
    return pl.kernel(
        body,
        out_type=jax.ShapeDtypeStruct((NW, N_PAD), jnp.float32),
        mesh=_mesh(),
        scratch_types=[
            pltpu.VMEM((NCHUNK, CHUNK), jnp.int32),
            pltpu.VMEM((N_PAD,), jnp.float32),
        ],
        compiler_params=cp,
    )


def _sc_shuf_gather():
    def body(h_hbm, idx_hbm, out_hbm, idx_v, rows_v):
        cid = jax.lax.axis_index("c")
        sid = jax.lax.axis_index("s")
        wid = cid * NS + sid
        pltpu.sync_copy(idx_hbm.at[wid], idx_v)

        @pl.loop(0, S_NCHUNK)
        def _(j):
            pltpu.sync_copy(h_hbm.at[idx_v.at[j]], rows_v)
            pltpu.sync_copy(
                rows_v, out_hbm.at[pl.ds(wid * SPT + j * S_CHUNK, S_CHUNK)])

    return pl.kernel(
        body,
        out_type=jax.ShapeDtypeStruct((SHUF_PAD, D), jnp.float32),
        mesh=_mesh(),
        scratch_types=[
            pltpu.VMEM((S_NCHUNK, S_CHUNK), jnp.int32),
            pltpu.VMEM((S_CHUNK, D), jnp.float32),
        ],
    )


def _tc_layer1_body(agg_ref, deg_ref, h_ref, w_ref, b_ref, o_ref):
    d = jnp.sum(deg_ref[...], axis=0)[:, None]
    t = (agg_ref[0] + agg_ref[1] - h_ref[...]) / (d + 1.0)
    y = jnp.dot(t, w_ref[...], preferred_element_type=jnp.float32)
    o_ref[...] = jnp.maximum(y + b_ref[...][None, :], 0.0)


def _tc_layer2_body(agg_ref, deg_ref, h_ref, w_ref, b_ref, wd_ref, bd_ref,
                    h2_ref, logp_ref):
    d = jnp.sum(deg_ref[...], axis=0)[:, None]
    t = (agg_ref[0] + agg_ref[1] - h_ref[...]) / (d + 1.0)
    h2 = jnp.dot(t, w_ref[...], preferred_element_type=jnp.float32)
    h2 = h2 + b_ref[...][None, :]
    h2_ref[...] = h2
    a = jnp.dot(h2, wd_ref[...], preferred_element_type=jnp.float32)
    a = a + bd_ref[...][None, :]
    m = jnp.max(a, axis=-1, keepdims=True)
    ex = jnp.exp(a - m)
    lse = jnp.log(jnp.sum(ex, axis=-1, keepdims=True))
    logp_ref[...] = a - m - lse


def _tc_dec_body(hs_ref, logp_ref, wd_ref, bd_ref, o_ref):
    hs = hs_ref[0:N_PAD, :]
    hb = jnp.dot(hs, wd_ref[...], preferred_element_type=jnp.float32)
    hb = hb + bd_ref[...][None, :]
    prod = hb * logp_ref[...]
    rid = jax.lax.broadcasted_iota(jnp.int32, (N_PAD, D), 0)
    s = jnp.sum(jnp.where(rid < N, prod, 0.0))
    o_ref[...] = jnp.broadcast_to(-s / N, (1, 1))


def kernel(x, edge_index, shuffled_index, W1, b1, W2, b2, Wd, bd):
    f32 = jnp.float32
    src = edge_index[0].astype(jnp.int32)
    dst = edge_index[1].astype(jnp.int32)

    pad = EPT - (E // NW)
    src_r = src.reshape(NW, E // NW)
    dst_r = dst.reshape(NW, E // NW)
    pad_src = jnp.zeros((NW, pad), jnp.int32)
    pad_dst = jnp.broadcast_to(
        N + (jnp.arange(pad, dtype=jnp.int32) % (N_PAD - N)), (NW, pad))
    src_p = jnp.concatenate([src_r, pad_src], axis=1).reshape(NW, NCHUNK, CHUNK)
    dst_p = jnp.concatenate([dst_r, pad_dst], axis=1).reshape(NW, NCHUNK, CHUNK)

    x_pad = jnp.zeros((N_PAD, D), f32).at[:N].set(x)

    shuf = shuffled_index.astype(jnp.int32)
    shuf_p = jnp.zeros((SHUF_PAD,), jnp.int32).at[:N].set(shuf)
    shuf_p = shuf_p.reshape(NW, S_NCHUNK, S_CHUNK)

    sc_agg = _sc_agg()
    sc_deg = _sc_deg()
    sc_gather = _sc_shuf_gather()

    deg = sc_deg(dst_p)
    agg1 = sc_agg(x_pad, src_p, dst_p)

    h1 = pl.pallas_call(
        _tc_layer1_body,
        out_shape=jax.ShapeDtypeStruct((N_PAD, D), f32),
    )(agg1, deg, x_pad, W1, b1)

    agg2 = sc_agg(h1, src_p, dst_p)

    h2, logp = pl.pallas_call(
        _tc_layer2_body,
        out_shape=(jax.ShapeDtypeStruct((N_PAD, D), f32),
                   jax.ShapeDtypeStruct((N_PAD, D), f32)),
    )(agg2, deg, h1, W2, b2, Wd, bd)

    hs = sc_gather(h2, shuf_p)

    dec = pl.pallas_call(
        _tc_dec_body,
        out_shape=jax.ShapeDtypeStruct((1, 1), f32),
    )(hs, logp, Wd, bd)

    return h2[:N], dec[0, 0]
